# baseline (speedup 1.0000x reference)
import jax
import jax.numpy as jnp
from jax.experimental import pallas as pl
from jax.experimental.pallas import tpu as pltpu

_K = 200
_D = 36
_KPAD = 256

_DIAG_OFF = []
_off = 0
for _r in range(_D):
    _DIAG_OFF.append(_off)
    _off += -(-(_D - _r) // 8) * 8
_LIN_OFF = _off
_CONST_OFF = _LIN_OFF + 40
_CC = _CONST_OFF + 8

_TNL = 2048
_LOG2PI = 1.8378770664093453


def _prep_kernel(covs_ref, means_ref, w_ref, out_ref):
    CT = covs_ref[...].reshape(_K, _D * _D).T
    mu = means_ref[...].T
    rowid = jax.lax.broadcasted_iota(jnp.int32, (_D, _K), 0)

    G = [CT[_D * j : _D * (j + 1), :] for j in range(_D)]
    Lcol = [None] * _D
    logdet = jnp.zeros((1, _K), jnp.float32)
    for j in range(_D):
        dj = G[j][j : j + 1, :]
        logdet = logdet + jnp.log(dj)
        colj = G[j] * jax.lax.rsqrt(dj)
        colj = jnp.where(rowid >= j, colj, 0.0)
        Lcol[j] = colj
        for i in range(j + 1, _D):
            G[i] = G[i] - colj[i : i + 1, :] * colj

    Vrow = []
    for i in range(_D):
        acc = jnp.where(rowid == i, 1.0, 0.0)
        for p in range(i):
            acc = acc - Lcol[p][i : i + 1, :] * Vrow[p]
        Vrow.append(acc / Lcol[i][i : i + 1, :])

    segs = []
    for r in range(_D):
        nrow = _D - r
        dr = Vrow[0][:nrow, :] * Vrow[0][r:, :]
        for p in range(1, _D):
            dr = dr + Vrow[p][:nrow, :] * Vrow[p][r:, :]
        if r > 0:
            dr = dr * 2.0
        pad = -nrow % 8
        if pad:
            dr = jnp.concatenate(
                [dr, jnp.zeros((pad, _K), jnp.float32)], axis=0)
        segs.append(dr)

    t = [jnp.sum(Vrow[p] * mu, axis=0, keepdims=True) for p in range(_D)]
    linslab = t[0] * Vrow[0]
    for p in range(1, _D):
        linslab = linslab + t[p] * Vrow[p]
    lin = -2.0 * linslab
    q = t[0] * t[0]
    for p in range(1, _D):
        q = q + t[p] * t[p]

    segs.append(lin)
    segs.append(jnp.zeros((4, _K), jnp.float32))

    c = logdet + _D * _LOG2PI - 2.0 * jnp.log(w_ref[...])
    qc = q + c
    qc_hi = qc.astype(jnp.bfloat16).astype(jnp.float32)
    segs.append(qc_hi)
    segs.append(qc - qc_hi)
    segs.append(jnp.zeros((6, _K), jnp.float32))

    Wl = jnp.concatenate(segs, axis=0)
    out_ref[...] = Wl.T.astype(jnp.bfloat16)


def _nll_kernel(xt_ref, w_ref, out_ref):
    nb, nc, _ = xt_ref.shape
    X = xt_ref[...].reshape(nb * nc, _D).T.astype(jnp.bfloat16)
    tnl = X.shape[1]
    segs = []
    for r in range(_D):
        nrow = _D - r
        prod = X[:nrow, :] * X[r:, :]
        pad = -nrow % 8
        if pad:
            prod = jnp.concatenate(
                [prod, jnp.zeros((pad, tnl), jnp.bfloat16)], axis=0)
        segs.append(prod)
    segs.append(X)
    segs.append(jnp.zeros((4, tnl), jnp.bfloat16))
    segs.append(jnp.ones((8, tnl), jnp.bfloat16))
    P = jnp.concatenate(segs, axis=0)
    m = jnp.dot(w_ref[...], P, preferred_element_type=jnp.float32)
    s = -0.5 * m
    smax = jnp.max(s, axis=0, keepdims=True)
    ll = jnp.log(jnp.sum(jnp.exp(s - smax), axis=0, keepdims=True)) + smax
    out_ref[...] = (-ll).reshape(nb, nc)


def kernel(x, means, covs, weights):
    B, P, d = x.shape

    Wb = pl.pallas_call(
        _prep_kernel,
        grid=(1,),
        in_specs=[
            pl.BlockSpec((_K, _D, _D), lambda i: (0, 0, 0)),
            pl.BlockSpec((_K, _D), lambda i: (0, 0)),
            pl.BlockSpec((1, _K), lambda i: (0, 0)),
        ],
        out_specs=pl.BlockSpec((_K, _CC), lambda i: (0, 0)),
        out_shape=jax.ShapeDtypeStruct((_K, _CC), jnp.bfloat16),
    )(covs, means, weights.reshape(1, _K))

    tc = _TNL // B
    out = pl.pallas_call(
        _nll_kernel,
        grid=(P // tc,),
        in_specs=[
            pl.BlockSpec((B, tc, _D), lambda i: (0, i, 0)),
            pl.BlockSpec((_K, _CC), lambda i: (0, 0)),
        ],
        out_specs=pl.BlockSpec((B, tc), lambda i: (0, i)),
        out_shape=jax.ShapeDtypeStruct((B, P), jnp.float32),
        compiler_params=pltpu.CompilerParams(
            dimension_semantics=("arbitrary",),
        ),
    )(x, Wb)
    return out


# speedup vs baseline: 1.0506x; 1.0506x over previous
import jax
import jax.numpy as jnp
from jax.experimental import pallas as pl
from jax.experimental.pallas import tpu as pltpu

_K = 200
_D = 36

_DIAG_OFF = []
_off = 0
for _r in range(_D):
    _DIAG_OFF.append(_off)
    _off += -(-(_D - _r) // 8) * 8
_LIN_OFF = _off
_CONST_OFF = _LIN_OFF + 40
_CC = _CONST_OFF + 8

_TNL = 2048
_LOG2PI = 1.8378770664093453


def _prep_kernel(covs_ref, means_ref, w_ref, out_ref):
    CT = covs_ref[...].T
    mu = means_ref[...].T
    rowid = jax.lax.broadcasted_iota(jnp.int32, (_D, _K), 0)

    G = [CT[_D * j : _D * (j + 1), :] for j in range(_D)]
    Lcol = [None] * _D
    logdet = jnp.zeros((1, _K), jnp.float32)
    for j in range(_D):
        dj = G[j][j : j + 1, :]
        logdet = logdet + jnp.log(dj)
        colj = G[j] * jax.lax.rsqrt(dj)
        colj = jnp.where(rowid >= j, colj, 0.0)
        Lcol[j] = colj
        for i in range(j + 1, _D):
            G[i] = G[i] - colj[i : i + 1, :] * colj

    Vrow = []
    for i in range(_D):
        acc = jnp.where(rowid == i, 1.0, 0.0)
        for p in range(i):
            acc = acc - Lcol[p][i : i + 1, :] * Vrow[p]
        Vrow.append(acc / Lcol[i][i : i + 1, :])

    segs = []
    for r in range(_D):
        nrow = _D - r
        dr = Vrow[0][:nrow, :] * Vrow[0][r:, :]
        for p in range(1, _D):
            dr = dr + Vrow[p][:nrow, :] * Vrow[p][r:, :]
        if r > 0:
            dr = dr * 2.0
        pad = -nrow % 8
        if pad:
            dr = jnp.concatenate(
                [dr, jnp.zeros((pad, _K), jnp.float32)], axis=0)
        segs.append(dr)

    t = [jnp.sum(Vrow[p] * mu, axis=0, keepdims=True) for p in range(_D)]
    linslab = t[0] * Vrow[0]
    for p in range(1, _D):
        linslab = linslab + t[p] * Vrow[p]
    lin = -2.0 * linslab
    q = t[0] * t[0]
    for p in range(1, _D):
        q = q + t[p] * t[p]

    segs.append(lin)
    segs.append(jnp.zeros((4, _K), jnp.float32))

    c = logdet + _D * _LOG2PI - 2.0 * jnp.log(w_ref[...])
    qc = q + c
    qc_hi = qc.astype(jnp.bfloat16).astype(jnp.float32)
    segs.append(qc_hi)
    segs.append(qc - qc_hi)
    segs.append(jnp.zeros((6, _K), jnp.float32))

    Wl = jnp.concatenate(segs, axis=0)
    out_ref[...] = Wl.T.astype(jnp.bfloat16)


def _nll_kernel(xt_ref, w_ref, out_ref):
    nb, nc, _ = xt_ref.shape
    X = xt_ref[...].reshape(nb * nc, _D).T.astype(jnp.bfloat16)
    tnl = X.shape[1]
    segs = []
    for r in range(_D):
        nrow = _D - r
        prod = X[:nrow, :] * X[r:, :]
        pad = -nrow % 8
        if pad:
            prod = jnp.concatenate(
                [prod, jnp.zeros((pad, tnl), jnp.bfloat16)], axis=0)
        segs.append(prod)
    segs.append(X)
    segs.append(jnp.zeros((4, tnl), jnp.bfloat16))
    segs.append(jnp.ones((8, tnl), jnp.bfloat16))
    P = jnp.concatenate(segs, axis=0)
    m = jnp.dot(w_ref[...], P, preferred_element_type=jnp.float32)
    s = -0.5 * m
    smax = jnp.max(s, axis=0, keepdims=True)
    ll = jnp.log(jnp.sum(jnp.exp(s - smax), axis=0, keepdims=True)) + smax
    out_ref[...] = (-ll).reshape(nb, nc)


def kernel(x, means, covs, weights):
    B, P, d = x.shape

    Wb = pl.pallas_call(
        _prep_kernel,
        grid=(1,),
        in_specs=[
            pl.BlockSpec((_K, _D * _D), lambda i: (0, 0)),
            pl.BlockSpec((_K, _D), lambda i: (0, 0)),
            pl.BlockSpec((1, _K), lambda i: (0, 0)),
        ],
        out_specs=pl.BlockSpec((_K, _CC), lambda i: (0, 0)),
        out_shape=jax.ShapeDtypeStruct((_K, _CC), jnp.bfloat16),
    )(covs.reshape(_K, _D * _D), means, weights.reshape(1, _K))

    tc = _TNL // B
    out = pl.pallas_call(
        _nll_kernel,
        grid=(P // tc,),
        in_specs=[
            pl.BlockSpec((B, tc, _D), lambda i: (0, i, 0)),
            pl.BlockSpec((_K, _CC), lambda i: (0, 0)),
        ],
        out_specs=pl.BlockSpec((B, tc), lambda i: (0, i)),
        out_shape=jax.ShapeDtypeStruct((B, P), jnp.float32),
        compiler_params=pltpu.CompilerParams(
            dimension_semantics=("arbitrary",),
        ),
    )(x, Wb)
    return out
